# baseline (speedup 1.0000x reference)
import jax
import jax.numpy as jnp
from jax.experimental import pallas as pl
from jax.experimental.pallas import tpu as pltpu

_RROWS = 1536
_TPAD = 1024


def _qsample_body(t_ref, ab_ref, x_ref, n_ref, o_ref, coef_ref):
    @pl.when(pl.program_id(0) == 0)
    def _():
        t_row = t_ref[...]
        ids = jax.lax.broadcasted_iota(jnp.int32, (_TPAD, t_row.shape[1]), 0)
        m = ids == t_row
        abt = jnp.transpose(ab_ref[...], (1, 0))
        zero = jnp.zeros((), jnp.float32)
        coef_ref[0:1, :] = jnp.sum(jnp.where(m, abt[:, 0:1], zero), axis=0, keepdims=True)
        coef_ref[1:2, :] = jnp.sum(jnp.where(m, abt[:, 1:2], zero), axis=0, keepdims=True)

    ca = coef_ref[0:1, :]
    cb = coef_ref[1:2, :]
    o_ref[...] = ca * x_ref[...] + cb * n_ref[...]


def kernel(x_start, t, noise, sqrt_alphas_cumprod, sqrt_one_minus_alphas_cumprod):
    B, C, H, W = x_start.shape
    P = C * H * W
    xt = x_start.transpose(1, 2, 3, 0).reshape(P, B)
    nt = noise.transpose(1, 2, 3, 0).reshape(P, B)
    t1 = t.reshape(1, B)
    T = sqrt_alphas_cumprod.shape[0]
    ab = (
        jnp.zeros((2, _TPAD), jnp.float32)
        .at[0, :T].set(sqrt_alphas_cumprod)
        .at[1, :T].set(sqrt_one_minus_alphas_cumprod)
    )

    out = pl.pallas_call(
        _qsample_body,
        grid=(P // _RROWS,),
        in_specs=[
            pl.BlockSpec((1, B), lambda i: (0, 0)),
            pl.BlockSpec((2, _TPAD), lambda i: (0, 0)),
            pl.BlockSpec((_RROWS, B), lambda i: (i, 0)),
            pl.BlockSpec((_RROWS, B), lambda i: (i, 0)),
        ],
        out_specs=pl.BlockSpec((_RROWS, B), lambda i: (i, 0)),
        out_shape=jax.ShapeDtypeStruct((P, B), jnp.float32),
        scratch_shapes=[pltpu.VMEM((2, B), jnp.float32)],
        compiler_params=pltpu.CompilerParams(dimension_semantics=("arbitrary",)),
    )(t1, ab, xt, nt)
    return out.reshape(C, H, W, B).transpose(3, 0, 1, 2)


# speedup vs baseline: 1.0028x; 1.0028x over previous
import jax
import jax.numpy as jnp
from jax.experimental import pallas as pl
from jax.experimental.pallas import tpu as pltpu

_RROWS = 2048
_TPAD = 1024


def _qsample_body(t_ref, ab_ref, x_ref, n_ref, o_ref, coef_ref):
    @pl.when(pl.program_id(0) == 0)
    def _():
        t_row = t_ref[...]
        ids = jax.lax.broadcasted_iota(jnp.int32, (_TPAD, t_row.shape[1]), 0)
        m = ids == t_row
        abt = jnp.transpose(ab_ref[...], (1, 0))
        zero = jnp.zeros((), jnp.float32)
        coef_ref[0:1, :] = jnp.sum(jnp.where(m, abt[:, 0:1], zero), axis=0, keepdims=True)
        coef_ref[1:2, :] = jnp.sum(jnp.where(m, abt[:, 1:2], zero), axis=0, keepdims=True)

    ca = coef_ref[0:1, :]
    cb = coef_ref[1:2, :]
    o_ref[...] = ca * x_ref[...] + cb * n_ref[...]


def kernel(x_start, t, noise, sqrt_alphas_cumprod, sqrt_one_minus_alphas_cumprod):
    B, C, H, W = x_start.shape
    P = C * H * W
    xt = x_start.transpose(1, 2, 3, 0).reshape(P, B)
    nt = noise.transpose(1, 2, 3, 0).reshape(P, B)
    t1 = t.reshape(1, B)
    T = sqrt_alphas_cumprod.shape[0]
    ab = (
        jnp.zeros((2, _TPAD), jnp.float32)
        .at[0, :T].set(sqrt_alphas_cumprod)
        .at[1, :T].set(sqrt_one_minus_alphas_cumprod)
    )

    out = pl.pallas_call(
        _qsample_body,
        grid=(P // _RROWS,),
        in_specs=[
            pl.BlockSpec((1, B), lambda i: (0, 0)),
            pl.BlockSpec((2, _TPAD), lambda i: (0, 0)),
            pl.BlockSpec((_RROWS, B), lambda i: (i, 0)),
            pl.BlockSpec((_RROWS, B), lambda i: (i, 0)),
        ],
        out_specs=pl.BlockSpec((_RROWS, B), lambda i: (i, 0)),
        out_shape=jax.ShapeDtypeStruct((P, B), jnp.float32),
        scratch_shapes=[pltpu.VMEM((2, B), jnp.float32)],
        compiler_params=pltpu.CompilerParams(dimension_semantics=("arbitrary",)),
    )(t1, ab, xt, nt)
    return out.reshape(C, H, W, B).transpose(3, 0, 1, 2)
